# Initial kernel scaffold; baseline (speedup 1.0000x reference)
import jax
import jax.numpy as jnp
from jax.experimental import pallas as pl


def kernel(x, edge_index, edge_weight, w1, b1, w2, b2):
    raise NotImplementedError("write your pallas kernel here")


# speedup vs baseline: 9.5005x; 9.5005x over previous
import functools

import jax
import jax.numpy as jnp
from jax import lax
from jax.experimental import pallas as pl
from jax.experimental.pallas import tpu as pltpu
from jax.experimental.pallas import tpu_sc as plsc

_NC = 2
_NS = 16
_NW = _NC * _NS
_CHUNK = 80


def _rsqrt16(d):
    i = lax.bitcast_convert_type(d, jnp.int32)
    i = jnp.int32(0x5F3759DF) - (i >> 1)
    y = lax.bitcast_convert_type(i, jnp.float32)
    for _ in range(3):
        y = y * (1.5 - 0.5 * d * y * y)
    return y


def _sc_body(n_nodes, n_pad, d, deg_chunks, edge_chunks, rows_per_tile,
             h_hbm, row_hbm, col_hbm, ew_hbm, out_hbm,
             deg_sh, s_sh, colv, eww, roww, drv, dcv, rows, zbuf, dbuf):
    c = lax.axis_index("c")
    s = lax.axis_index("s")
    wid = s * _NC + c
    zrows = zbuf.shape[0]

    def zrow(i, _):
        for j in range(d // 16):
            zbuf[i, pl.ds(j * 16, 16)] = jnp.zeros((16,), jnp.float32)
        return 0
    lax.fori_loop(0, zrows, zrow, 0)
    for k in range(rows_per_tile // zrows):
        pltpu.sync_copy(zbuf, s_sh.at[pl.ds(s * rows_per_tile + k * zrows, zrows)])
    dvals = n_pad // _NS
    def zd(i, _):
        dbuf[pl.ds(i * 16, 16)] = jnp.zeros((16,), jnp.float32)
        return 0
    lax.fori_loop(0, dvals // 16, zd, 0)
    pltpu.sync_copy(dbuf, deg_sh.at[pl.ds(s * dvals, dvals)])
    plsc.subcore_barrier()

    def dstep(i, _):
        base = (s * deg_chunks + i) * _CHUNK
        pltpu.sync_copy(col_hbm.at[pl.ds(base, _CHUNK)], colv)
        pltpu.sync_copy(ew_hbm.at[pl.ds(base, _CHUNK)], eww.at[pl.ds(0, _CHUNK)])
        pltpu.sync_copy(eww.at[pl.ds(0, _CHUNK)], deg_sh.at[colv], add=True)
        return 0
    lax.fori_loop(0, deg_chunks, dstep, 0)
    plsc.subcore_barrier()

    pltpu.sync_copy(deg_sh.at[pl.ds(s * dvals, dvals)], dbuf)
    def rstep(j, _):
        dd = dbuf[pl.ds(j * 16, 16)]
        dbuf[pl.ds(j * 16, 16)] = _rsqrt16(dd)
        return 0
    lax.fori_loop(0, dvals // 16, rstep, 0)
    pltpu.sync_copy(dbuf, deg_sh.at[pl.ds(s * dvals, dvals)])
    plsc.subcore_barrier()

    def estep(i, _):
        base = (wid * edge_chunks + i) * _CHUNK
        pltpu.sync_copy(row_hbm.at[pl.ds(base, _CHUNK)], roww)
        pltpu.sync_copy(col_hbm.at[pl.ds(base, _CHUNK)], colv)
        pltpu.sync_copy(ew_hbm.at[pl.ds(base, _CHUNK)], eww.at[pl.ds(0, _CHUNK)])
        pltpu.sync_copy(deg_sh.at[roww], drv)
        pltpu.sync_copy(deg_sh.at[colv], dcv)
        for j in range(_CHUNK // 16):
            sl = pl.ds(j * 16, 16)
            eww[sl] = eww[sl] * drv[sl] * dcv[sl]
        pltpu.sync_copy(h_hbm.at[roww], rows)
        def scale_row(e, _):
            v = eww[pl.ds(e, 16)]
            w = jnp.broadcast_to(v[0], (16,))
            for j in range(d // 16):
                sl = pl.ds(j * 16, 16)
                rows[e, sl] = rows[e, sl] * w
            return 0
        lax.fori_loop(0, _CHUNK, scale_row, 0)
        pltpu.sync_copy(rows, s_sh.at[colv], add=True)
        return 0
    lax.fori_loop(0, edge_chunks, estep, 0)
    plsc.subcore_barrier()

    stripe = (n_nodes // _NS) // 8 * 8
    pltpu.sync_copy(s_sh.at[pl.ds(s * stripe, stripe)],
                    out_hbm.at[c, pl.ds(s * stripe, stripe)])
    tail = n_nodes - stripe * _NS
    if tail:
        @pl.when(s == 0)
        def _():
            pltpu.sync_copy(s_sh.at[pl.ds(stripe * _NS, tail)],
                            out_hbm.at[c, pl.ds(stripe * _NS, tail)])


def _mm_body(x_ref, w_ref, o_ref):
    o_ref[...] = jnp.dot(x_ref[...], w_ref[...], preferred_element_type=jnp.float32)


def _out_body(s_ref, b1_ref, w2_ref, b2_ref, o_ref):
    t = s_ref[0] + s_ref[1] + b1_ref[...]
    t = jnp.maximum(t, 0.0)
    o_ref[...] = jnp.dot(t, w2_ref[...], preferred_element_type=jnp.float32) + b2_ref[...]


def kernel(x, edge_index, edge_weight, w1, b1, w2, b2):
    n, d_in = x.shape
    e = edge_weight.shape[0]
    d_hid = w1.shape[1]
    d_out = w2.shape[1]

    h = pl.pallas_call(
        _mm_body,
        out_shape=jax.ShapeDtypeStruct((n, d_hid), jnp.float32),
    )(x, w1)

    e_aug = e + n
    per_w = -(-e_aug // (_NW * _CHUNK)) * _CHUNK
    e_pad = per_w * _NW
    loop_idx = jnp.arange(n, dtype=jnp.int32)
    pad = e_pad - e_aug
    row = jnp.concatenate([edge_index[0], loop_idx, jnp.zeros((pad,), jnp.int32)])
    col = jnp.concatenate([edge_index[1], loop_idx, jnp.zeros((pad,), jnp.int32)])
    ew = jnp.concatenate([edge_weight, jnp.ones((n,), jnp.float32),
                          jnp.zeros((pad,), jnp.float32)])

    n_pad = -(-n // (_NS * 16)) * (_NS * 16)
    deg_chunks = e_pad // (_NS * _CHUNK)
    edge_chunks = per_w // _CHUNK
    rows_per_tile = n // _NS
    assert rows_per_tile * _NS == n and rows_per_tile % 125 == 0

    mesh = plsc.VectorSubcoreMesh(core_axis_name="c", subcore_axis_name="s",
                                  num_cores=_NC, num_subcores=_NS)
    body = functools.partial(_sc_body, n, n_pad, d_hid, deg_chunks, edge_chunks,
                             rows_per_tile)
    s_partial = pl.kernel(
        body,
        out_type=jax.ShapeDtypeStruct((_NC, n, d_hid), jnp.float32),
        mesh=mesh,
        scratch_types=[
            pltpu.VMEM_SHARED((n_pad,), jnp.float32),
            pltpu.VMEM_SHARED((n, d_hid), jnp.float32),
            pltpu.VMEM((_CHUNK,), jnp.int32),
            pltpu.VMEM((_CHUNK + 16,), jnp.float32),
            pltpu.VMEM((_CHUNK,), jnp.int32),
            pltpu.VMEM((_CHUNK,), jnp.float32),
            pltpu.VMEM((_CHUNK,), jnp.float32),
            pltpu.VMEM((_CHUNK, d_hid), jnp.float32),
            pltpu.VMEM((125, d_hid), jnp.float32),
            pltpu.VMEM((n_pad // _NS,), jnp.float32),
        ],
    )(h, row, col, ew)

    out = pl.pallas_call(
        _out_body,
        out_shape=jax.ShapeDtypeStruct((n, d_out), jnp.float32),
    )(s_partial, b1.reshape(1, d_hid), w2, b2.reshape(1, d_out))
    return out


# speedup vs baseline: 31.8274x; 3.3501x over previous
import functools

import jax
import jax.numpy as jnp
from jax import lax
from jax.experimental import pallas as pl
from jax.experimental.pallas import tpu as pltpu
from jax.experimental.pallas import tpu_sc as plsc

_NC = 2
_NS = 16
_NW = _NC * _NS
_CHUNK = 96
_RING = 3
_RING2 = 6


def _rsqrt16(d):
    i = lax.bitcast_convert_type(d, jnp.int32)
    i = jnp.int32(0x5F3759DF) - (i >> 1)
    y = lax.bitcast_convert_type(i, jnp.float32)
    for _ in range(3):
        y = y * (1.5 - 0.5 * d * y * y)
    return y


def _sc_body(n_nodes, n_pad, d, nch,
             h_hbm, row_hbm, col_hbm, ew_hbm, out_hbm,
             deg_sh, s_sh, sbuf, zbuf, dbuf,
             rowb, colb, ewb, drvb, dcvb, rowsb,
             sem_i, sem_g, sem_d, sem_s, sem_ds):
    c = lax.axis_index("c")
    s = lax.axis_index("s")
    wid = s * _NC + c
    deg_nch = _NC * nch
    nvec = d // 16

    zrows = zbuf.shape[0]
    def zrow(i, _):
        for j in range(nvec):
            zbuf[i, pl.ds(j * 16, 16)] = jnp.zeros((16,), jnp.float32)
        return 0
    lax.fori_loop(0, zrows, zrow, 0)
    rows_per_tile = n_nodes // _NS
    for k in range(rows_per_tile // zrows):
        pltpu.sync_copy(zbuf, s_sh.at[pl.ds(s * rows_per_tile + k * zrows, zrows)])
    dvals = n_pad // _NS
    def zd(i, _):
        dbuf[pl.ds(i * 16, 16)] = jnp.zeros((16,), jnp.float32)
        return 0
    lax.fori_loop(0, dvals // 16, zd, 0)
    pltpu.sync_copy(dbuf, deg_sh.at[pl.ds(s * dvals, dvals)])
    plsc.subcore_barrier()

    def dg_loads(j, q):
        base = (s * deg_nch + j) * _CHUNK
        pltpu.async_copy(col_hbm.at[pl.ds(base, _CHUNK)], colb[q], sem_i[q])
        pltpu.async_copy(ew_hbm.at[pl.ds(base, _CHUNK)], ewb[q], sem_i[q])
    def dg_wait_loads(j, q):
        base = (s * deg_nch + j) * _CHUNK
        pltpu.make_async_copy(col_hbm.at[pl.ds(base, _CHUNK)], colb[q], sem_i[q]).wait()
        pltpu.make_async_copy(ew_hbm.at[pl.ds(base, _CHUNK)], ewb[q], sem_i[q]).wait()
    for q in range(3):
        dg_loads(q, q)

    def dstep(p, _):
        for q in range(_RING2):
            j = p * _RING2 + q
            q3 = (q + 3) % _RING2
            dg_wait_loads(j, q)
            pltpu.async_copy(ewb[q], deg_sh.at[colb[q]], sem_ds[q], add=True)
            nj = j + 3
            @pl.when(nj < deg_nch)
            def _():
                @pl.when(j >= 3)
                def _():
                    pltpu.make_async_copy(ewb[q3], deg_sh.at[colb[q3]], sem_ds[q3]).wait()
                dg_loads(nj, q3)
        return 0
    lax.fori_loop(0, deg_nch // _RING2, dstep, 0)
    for q in range(_RING2):
        pltpu.make_async_copy(ewb[q], deg_sh.at[colb[q]], sem_ds[q]).wait()
    plsc.subcore_barrier()

    pltpu.sync_copy(deg_sh.at[pl.ds(s * dvals, dvals)], dbuf)
    def rstep(j, _):
        dd = dbuf[pl.ds(j * 16, 16)]
        dbuf[pl.ds(j * 16, 16)] = _rsqrt16(dd)
        return 0
    lax.fori_loop(0, dvals // 16, rstep, 0)
    pltpu.sync_copy(dbuf, deg_sh.at[pl.ds(s * dvals, dvals)])
    plsc.subcore_barrier()

    def mp_loads(j, q):
        base = (wid * nch + j) * _CHUNK
        pltpu.async_copy(row_hbm.at[pl.ds(base, _CHUNK)], rowb[q], sem_i[q])
        pltpu.async_copy(col_hbm.at[pl.ds(base, _CHUNK)], colb[q], sem_i[q])
        pltpu.async_copy(ew_hbm.at[pl.ds(base, _CHUNK)], ewb[q], sem_i[q])

    def mp_wait_loads(j, q):
        base = (wid * nch + j) * _CHUNK
        pltpu.make_async_copy(row_hbm.at[pl.ds(base, _CHUNK)], rowb[q], sem_i[q]).wait()
        pltpu.make_async_copy(col_hbm.at[pl.ds(base, _CHUNK)], colb[q], sem_i[q]).wait()
        pltpu.make_async_copy(ew_hbm.at[pl.ds(base, _CHUNK)], ewb[q], sem_i[q]).wait()

    def mp_gathers(qi, qg):
        pltpu.async_copy(h_hbm.at[rowb[qi]], rowsb[qg], sem_g[qg])
        pltpu.async_copy(deg_sh.at[rowb[qi]], drvb[qg], sem_d[qg])
        pltpu.async_copy(deg_sh.at[colb[qi]], dcvb[qg], sem_d[qg])

    for q in range(4):
        mp_loads(q, q)
    mp_wait_loads(0, 0)
    mp_gathers(0, 0)

    def pstep(p, _):
        for q in range(_RING2):
            j = p * _RING2 + q
            qg = q % _RING
            q1i = (q + 1) % _RING2
            q1g = (q + 1) % _RING
            q4i = (q + 4) % _RING2
            @pl.when(j + 1 < nch)
            def _():
                mp_wait_loads(j + 1, q1i)
                @pl.when(j >= _RING - 1)
                def _():
                    pltpu.make_async_copy(rowsb[q1g],
                                          s_sh.at[colb[q4i]], sem_s[q1g]).wait()
                mp_gathers(q1i, q1g)
            pltpu.make_async_copy(deg_sh.at[rowb[q]], drvb[qg], sem_d[qg]).wait()
            pltpu.make_async_copy(deg_sh.at[colb[q]], dcvb[qg], sem_d[qg]).wait()
            for t in range(_CHUNK // 16):
                sl = pl.ds(t * 16, 16)
                sbuf[sl] = ewb[q][sl] * drvb[qg][sl] * dcvb[qg][sl]
            pltpu.make_async_copy(h_hbm.at[rowb[q]], rowsb[qg], sem_g[qg]).wait()
            rq = rowsb[qg]
            def scale8(t, _):
                for u in range(8):
                    ee = t * 8 + u
                    w = jnp.broadcast_to(sbuf[pl.ds(ee, 16)][0], (16,))
                    for v in range(nvec):
                        sl = pl.ds(v * 16, 16)
                        rq[ee, sl] = rq[ee, sl] * w
                return 0
            lax.fori_loop(0, _CHUNK // 8, scale8, 0)
            pltpu.async_copy(rq, s_sh.at[colb[q]], sem_s[qg], add=True)
            nj = j + 4
            @pl.when(nj < nch)
            def _():
                mp_loads(nj, q4i)
        return 0
    lax.fori_loop(0, nch // _RING2, pstep, 0)
    for q in range(_RING):
        qq = (nch - _RING + q) % _RING2
        pltpu.make_async_copy(rowsb[(nch - _RING + q) % _RING],
                              s_sh.at[colb[qq]], sem_s[(nch - _RING + q) % _RING]).wait()
    plsc.subcore_barrier()

    stripe = rows_per_tile // 8 * 8
    pltpu.sync_copy(s_sh.at[pl.ds(s * stripe, stripe)],
                    out_hbm.at[c, pl.ds(s * stripe, stripe)])
    tail = n_nodes - stripe * _NS
    if tail:
        @pl.when(s == 0)
        def _():
            pltpu.sync_copy(s_sh.at[pl.ds(stripe * _NS, tail)],
                            out_hbm.at[c, pl.ds(stripe * _NS, tail)])


def _mm_body(x_ref, w_ref, o_ref):
    o_ref[...] = jnp.dot(x_ref[...], w_ref[...], preferred_element_type=jnp.float32)


def _out_body(s_ref, b1_ref, w2_ref, b2_ref, o_ref):
    t = s_ref[0] + s_ref[1] + b1_ref[...]
    t = jnp.maximum(t, 0.0)
    o_ref[...] = jnp.dot(t, w2_ref[...], preferred_element_type=jnp.float32) + b2_ref[...]


def kernel(x, edge_index, edge_weight, w1, b1, w2, b2):
    n, d_in = x.shape
    e = edge_weight.shape[0]
    d_hid = w1.shape[1]
    d_out = w2.shape[1]

    h = pl.pallas_call(
        _mm_body,
        out_shape=jax.ShapeDtypeStruct((n, d_hid), jnp.float32),
    )(x, w1)

    e_aug = e + n
    nch = -(-e_aug // (_NW * _CHUNK))
    nch = -(-nch // _RING2) * _RING2
    per_w = nch * _CHUNK
    e_pad = per_w * _NW
    pad = e_pad - e_aug
    loop_idx = jnp.arange(n, dtype=jnp.int32)
    pad_idx = jnp.arange(pad, dtype=jnp.int32) % n
    row = jnp.concatenate([edge_index[0], loop_idx, pad_idx])
    col = jnp.concatenate([edge_index[1], loop_idx, pad_idx])
    ew = jnp.concatenate([edge_weight, jnp.ones((n,), jnp.float32),
                          jnp.zeros((pad,), jnp.float32)])

    n_pad = -(-n // (_NS * 16)) * (_NS * 16)
    rows_per_tile = n // _NS
    assert rows_per_tile * _NS == n and rows_per_tile % 25 == 0

    mesh = plsc.VectorSubcoreMesh(core_axis_name="c", subcore_axis_name="s",
                                  num_cores=_NC, num_subcores=_NS)
    body = functools.partial(_sc_body, n, n_pad, d_hid, nch)
    s_partial = pl.kernel(
        body,
        out_type=jax.ShapeDtypeStruct((_NC, n, d_hid), jnp.float32),
        mesh=mesh,
        scratch_types=[
            pltpu.VMEM_SHARED((n_pad,), jnp.float32),
            pltpu.VMEM_SHARED((n, d_hid), jnp.float32),
            pltpu.VMEM((_CHUNK + 16,), jnp.float32),
            pltpu.VMEM((25, d_hid), jnp.float32),
            pltpu.VMEM((n_pad // _NS,), jnp.float32),
            [pltpu.VMEM((_CHUNK,), jnp.int32)] * _RING2,
            [pltpu.VMEM((_CHUNK,), jnp.int32)] * _RING2,
            [pltpu.VMEM((_CHUNK,), jnp.float32)] * _RING2,
            [pltpu.VMEM((_CHUNK,), jnp.float32)] * _RING,
            [pltpu.VMEM((_CHUNK,), jnp.float32)] * _RING,
            [pltpu.VMEM((_CHUNK, d_hid), jnp.float32)] * _RING,
            [pltpu.SemaphoreType.DMA] * _RING2,
            [pltpu.SemaphoreType.DMA] * _RING,
            [pltpu.SemaphoreType.DMA] * _RING,
            [pltpu.SemaphoreType.DMA] * _RING,
            [pltpu.SemaphoreType.DMA] * _RING2,
        ],
    )(h, row, col, ew)

    out = pl.pallas_call(
        _out_body,
        out_shape=jax.ShapeDtypeStruct((n, d_out), jnp.float32),
    )(s_partial, b1.reshape(1, d_hid), w2, b2.reshape(1, d_out))
    return out


# speedup vs baseline: 38.1670x; 1.1992x over previous
import functools

import jax
import jax.numpy as jnp
from jax import lax
from jax.experimental import pallas as pl
from jax.experimental.pallas import tpu as pltpu
from jax.experimental.pallas import tpu_sc as plsc

_NC = 2
_NS = 16
_NW = _NC * _NS
_CHUNK = 96
_RING = 3
_RING2 = 6


def _rsqrt16(d):
    i = lax.bitcast_convert_type(d, jnp.int32)
    i = jnp.int32(0x5F3759DF) - (i >> 1)
    y = lax.bitcast_convert_type(i, jnp.float32)
    for _ in range(3):
        y = y * (1.5 - 0.5 * d * y * y)
    return y


def _deg_body(n_pad, nch,
              col_hbm, ew_hbm, out_hbm,
              deg_sh, dbuf, colb, ewb, sem_i, sem_ds):
    c = lax.axis_index("c")
    s = lax.axis_index("s")
    wid = s * _NC + c

    dvals = n_pad // _NS
    def zd(i, _):
        dbuf[pl.ds(i * 16, 16)] = jnp.zeros((16,), jnp.float32)
        return 0
    lax.fori_loop(0, dvals // 16, zd, 0)
    pltpu.sync_copy(dbuf, deg_sh.at[pl.ds(s * dvals, dvals)])
    plsc.subcore_barrier()

    def dg_loads(j, q):
        base = (wid * nch + j) * _CHUNK
        pltpu.async_copy(col_hbm.at[pl.ds(base, _CHUNK)], colb[q], sem_i[q])
        pltpu.async_copy(ew_hbm.at[pl.ds(base, _CHUNK)], ewb[q], sem_i[q])
    def dg_wait_loads(j, q):
        base = (wid * nch + j) * _CHUNK
        pltpu.make_async_copy(col_hbm.at[pl.ds(base, _CHUNK)], colb[q], sem_i[q]).wait()
        pltpu.make_async_copy(ew_hbm.at[pl.ds(base, _CHUNK)], ewb[q], sem_i[q]).wait()
    for q in range(3):
        dg_loads(q, q)

    def dstep(p, _):
        for q in range(_RING2):
            j = p * _RING2 + q
            q3 = (q + 3) % _RING2
            dg_wait_loads(j, q)
            pltpu.async_copy(ewb[q], deg_sh.at[colb[q]], sem_ds[q], add=True)
            nj = j + 3
            @pl.when(nj < nch)
            def _():
                @pl.when(j >= 3)
                def _():
                    pltpu.make_async_copy(ewb[q3], deg_sh.at[colb[q3]], sem_ds[q3]).wait()
                dg_loads(nj, q3)
        return 0
    lax.fori_loop(0, nch // _RING2, dstep, 0)
    for q in range(_RING2):
        pltpu.make_async_copy(ewb[q], deg_sh.at[colb[q]], sem_ds[q]).wait()
    plsc.subcore_barrier()

    pltpu.sync_copy(deg_sh.at[pl.ds(s * dvals, dvals)],
                    out_hbm.at[c, pl.ds(s * dvals, dvals)])


def _msg_body(n_nodes, n_pad, d, nch,
              h_hbm, row_hbm, col_hbm, ew_hbm, degp_hbm, out_hbm,
              deg_sh, s_sh, sbuf, zbuf, dbuf, dbuf2,
              rowb, colb, ewb, drvb, dcvb, rowsb,
              sem_i, sem_g, sem_d, sem_s):
    c = lax.axis_index("c")
    s = lax.axis_index("s")
    wid = s * _NC + c
    nvec = d // 16

    zrows = zbuf.shape[0]
    def zrow(i, _):
        for j in range(nvec):
            zbuf[i, pl.ds(j * 16, 16)] = jnp.zeros((16,), jnp.float32)
        return 0
    lax.fori_loop(0, zrows, zrow, 0)
    rows_per_tile = n_nodes // _NS
    for k in range(rows_per_tile // zrows):
        pltpu.sync_copy(zbuf, s_sh.at[pl.ds(s * rows_per_tile + k * zrows, zrows)])

    dvals = n_pad // _NS
    pltpu.sync_copy(degp_hbm.at[0, pl.ds(s * dvals, dvals)], dbuf)
    pltpu.sync_copy(degp_hbm.at[1, pl.ds(s * dvals, dvals)], dbuf2)
    def rstep(j, _):
        sl = pl.ds(j * 16, 16)
        dbuf[sl] = _rsqrt16(dbuf[sl] + dbuf2[sl])
        return 0
    lax.fori_loop(0, dvals // 16, rstep, 0)
    pltpu.sync_copy(dbuf, deg_sh.at[pl.ds(s * dvals, dvals)])
    plsc.subcore_barrier()

    def mp_loads(j, q):
        base = (wid * nch + j) * _CHUNK
        pltpu.async_copy(row_hbm.at[pl.ds(base, _CHUNK)], rowb[q], sem_i[q])
        pltpu.async_copy(col_hbm.at[pl.ds(base, _CHUNK)], colb[q], sem_i[q])
        pltpu.async_copy(ew_hbm.at[pl.ds(base, _CHUNK)], ewb[q], sem_i[q])

    def mp_wait_loads(j, q):
        base = (wid * nch + j) * _CHUNK
        pltpu.make_async_copy(row_hbm.at[pl.ds(base, _CHUNK)], rowb[q], sem_i[q]).wait()
        pltpu.make_async_copy(col_hbm.at[pl.ds(base, _CHUNK)], colb[q], sem_i[q]).wait()
        pltpu.make_async_copy(ew_hbm.at[pl.ds(base, _CHUNK)], ewb[q], sem_i[q]).wait()

    def mp_gathers(qi, qg):
        pltpu.async_copy(h_hbm.at[rowb[qi]], rowsb[qg], sem_g[qg])
        pltpu.async_copy(deg_sh.at[rowb[qi]], drvb[qg], sem_d[qg])
        pltpu.async_copy(deg_sh.at[colb[qi]], dcvb[qg], sem_d[qg])

    for q in range(4):
        mp_loads(q, q)
    mp_wait_loads(0, 0)
    mp_gathers(0, 0)

    def pstep(p, _):
        for q in range(_RING2):
            j = p * _RING2 + q
            qg = q % _RING
            q1i = (q + 1) % _RING2
            q1g = (q + 1) % _RING
            q4i = (q + 4) % _RING2
            @pl.when(j + 1 < nch)
            def _():
                mp_wait_loads(j + 1, q1i)
                @pl.when(j >= _RING - 1)
                def _():
                    pltpu.make_async_copy(rowsb[q1g],
                                          s_sh.at[colb[q4i]], sem_s[q1g]).wait()
                mp_gathers(q1i, q1g)
            pltpu.make_async_copy(deg_sh.at[rowb[q]], drvb[qg], sem_d[qg]).wait()
            pltpu.make_async_copy(deg_sh.at[colb[q]], dcvb[qg], sem_d[qg]).wait()
            for t in range(_CHUNK // 16):
                sl = pl.ds(t * 16, 16)
                sbuf[sl] = ewb[q][sl] * drvb[qg][sl] * dcvb[qg][sl]
            pltpu.make_async_copy(h_hbm.at[rowb[q]], rowsb[qg], sem_g[qg]).wait()
            rq = rowsb[qg]
            def scale16(g, _):
                vec = sbuf[pl.ds(g * 16, 16)]
                for u in range(16):
                    ee = g * 16 + u
                    w = jnp.broadcast_to(vec[u], (16,))
                    for v in range(nvec):
                        sl = pl.ds(v * 16, 16)
                        rq[ee, sl] = rq[ee, sl] * w
                return 0
            lax.fori_loop(0, _CHUNK // 16, scale16, 0)
            pltpu.async_copy(rq, s_sh.at[colb[q]], sem_s[qg], add=True)
            nj = j + 4
            @pl.when(nj < nch)
            def _():
                mp_loads(nj, q4i)
        return 0
    lax.fori_loop(0, nch // _RING2, pstep, 0)
    for q in range(_RING):
        qq = (nch - _RING + q) % _RING2
        pltpu.make_async_copy(rowsb[(nch - _RING + q) % _RING],
                              s_sh.at[colb[qq]], sem_s[(nch - _RING + q) % _RING]).wait()
    plsc.subcore_barrier()

    stripe = rows_per_tile // 8 * 8
    pltpu.sync_copy(s_sh.at[pl.ds(s * stripe, stripe)],
                    out_hbm.at[c, pl.ds(s * stripe, stripe)])
    tail = n_nodes - stripe * _NS
    if tail:
        @pl.when(s == 0)
        def _():
            pltpu.sync_copy(s_sh.at[pl.ds(stripe * _NS, tail)],
                            out_hbm.at[c, pl.ds(stripe * _NS, tail)])


def _mm_body(x_ref, w_ref, o_ref):
    o_ref[...] = jnp.dot(x_ref[...], w_ref[...], preferred_element_type=jnp.float32)


def _out_body(s_ref, b1_ref, w2_ref, b2_ref, o_ref):
    t = s_ref[0] + s_ref[1] + b1_ref[...]
    t = jnp.maximum(t, 0.0)
    o_ref[...] = jnp.dot(t, w2_ref[...], preferred_element_type=jnp.float32) + b2_ref[...]


def kernel(x, edge_index, edge_weight, w1, b1, w2, b2):
    n, d_in = x.shape
    e = edge_weight.shape[0]
    d_hid = w1.shape[1]
    d_out = w2.shape[1]

    e_aug = e + n
    nch = -(-e_aug // (_NW * _CHUNK))
    nch = -(-nch // _RING2) * _RING2
    per_w = nch * _CHUNK
    e_pad = per_w * _NW
    pad = e_pad - e_aug
    loop_idx = jnp.arange(n, dtype=jnp.int32)
    pad_idx = jnp.arange(pad, dtype=jnp.int32) % n
    row = jnp.concatenate([edge_index[0], loop_idx, pad_idx])
    col = jnp.concatenate([edge_index[1], loop_idx, pad_idx])
    ew = jnp.concatenate([edge_weight, jnp.ones((n,), jnp.float32),
                          jnp.zeros((pad,), jnp.float32)])

    n_pad = -(-n // (_NS * 16)) * (_NS * 16)
    rows_per_tile = n // _NS
    assert rows_per_tile * _NS == n and rows_per_tile % 25 == 0

    mesh = plsc.VectorSubcoreMesh(core_axis_name="c", subcore_axis_name="s",
                                  num_cores=_NC, num_subcores=_NS)

    degp = pl.kernel(
        functools.partial(_deg_body, n_pad, nch),
        out_type=jax.ShapeDtypeStruct((_NC, n_pad), jnp.float32),
        mesh=mesh,
        scratch_types=[
            pltpu.VMEM_SHARED((n_pad,), jnp.float32),
            pltpu.VMEM((n_pad // _NS,), jnp.float32),
            [pltpu.VMEM((_CHUNK,), jnp.int32)] * _RING2,
            [pltpu.VMEM((_CHUNK,), jnp.float32)] * _RING2,
            [pltpu.SemaphoreType.DMA] * _RING2,
            [pltpu.SemaphoreType.DMA] * _RING2,
        ],
    )(col, ew)

    h = pl.pallas_call(
        _mm_body,
        out_shape=jax.ShapeDtypeStruct((n, d_hid), jnp.float32),
    )(x, w1)

    s_partial = pl.kernel(
        functools.partial(_msg_body, n, n_pad, d_hid, nch),
        out_type=jax.ShapeDtypeStruct((_NC, n, d_hid), jnp.float32),
        mesh=mesh,
        scratch_types=[
            pltpu.VMEM_SHARED((n_pad,), jnp.float32),
            pltpu.VMEM_SHARED((n, d_hid), jnp.float32),
            pltpu.VMEM((_CHUNK + 16,), jnp.float32),
            pltpu.VMEM((25, d_hid), jnp.float32),
            pltpu.VMEM((n_pad // _NS,), jnp.float32),
            pltpu.VMEM((n_pad // _NS,), jnp.float32),
            [pltpu.VMEM((_CHUNK,), jnp.int32)] * _RING2,
            [pltpu.VMEM((_CHUNK,), jnp.int32)] * _RING2,
            [pltpu.VMEM((_CHUNK,), jnp.float32)] * _RING2,
            [pltpu.VMEM((_CHUNK,), jnp.float32)] * _RING,
            [pltpu.VMEM((_CHUNK,), jnp.float32)] * _RING,
            [pltpu.VMEM((_CHUNK, d_hid), jnp.float32)] * _RING,
            [pltpu.SemaphoreType.DMA] * _RING2,
            [pltpu.SemaphoreType.DMA] * _RING,
            [pltpu.SemaphoreType.DMA] * _RING,
            [pltpu.SemaphoreType.DMA] * _RING,
        ],
    )(h, row, col, ew, degp)

    out = pl.pallas_call(
        _out_body,
        out_shape=jax.ShapeDtypeStruct((n, d_out), jnp.float32),
    )(s_partial, b1.reshape(1, d_hid), w2, b2.reshape(1, d_out))
    return out
